# Initial kernel scaffold; baseline (speedup 1.0000x reference)
import jax
import jax.numpy as jnp
from jax.experimental import pallas as pl


def kernel(x, edge_index, W1, b1, W2, b2, Wc, bc):
    raise NotImplementedError("write your pallas kernel here")


# speedup vs baseline: 7.5418x; 7.5418x over previous
import functools

import jax
import jax.numpy as jnp
from jax import lax
from jax.experimental import pallas as pl
from jax.experimental.pallas import tpu as pltpu
from jax.experimental.pallas import tpu_sc as plsc

N = 10000
NP = 10240
E = 160000
D = 256
H = 128

EPT = E // 16
CH = 80
NCH = EPT // CH
RPT = NP // 16

def _mesh():
    return plsc.VectorSubcoreMesh(core_axis_name="c", subcore_axis_name="s",
                                  num_cores=2, num_subcores=16)


def _deg_body(dst_hbm, zeros_hbm, ones_hbm, out_hbm, acc, idx_v, ones_v):
    c = lax.axis_index("c")
    s = lax.axis_index("s")
    rows = pl.ds(s * RPT, RPT)
    pltpu.sync_copy(ones_hbm, ones_v)
    pltpu.sync_copy(zeros_hbm, acc.at[rows])
    plsc.subcore_barrier()
    base0 = c * (E // 2) + s * 5040
    nch = jnp.where(s == 15, 55, 63)

    def body(i, carry):
        pltpu.sync_copy(dst_hbm.at[pl.ds(base0 + i * CH, CH)], idx_v)
        pltpu.sync_copy(ones_v, acc.at[idx_v], add=True)
        return carry

    lax.fori_loop(0, nch, body, 0)
    plsc.subcore_barrier()
    pltpu.sync_copy(acc.at[rows], out_hbm.at[c].at[rows])


def _sc_degree(dst):
    zeros = jnp.zeros((RPT, H), jnp.float32)
    ones = jnp.ones((CH, H), jnp.float32)
    return pl.kernel(
        _deg_body,
        out_type=jax.ShapeDtypeStruct((2, NP, H), jnp.float32),
        mesh=_mesh(),
        scratch_types=[
            pltpu.VMEM_SHARED((NP, H), jnp.float32),
            pltpu.VMEM((CH,), jnp.int32),
            pltpu.VMEM((CH, H), jnp.float32),
        ],
    )(dst, zeros, ones)


def _scatter_body(ylo, yhi, src_hbm, dst_hbm, zeros_hbm, zlo_out, zhi_out,
                  zacc, src_v, dst_v, rows_v, sem):
    c = lax.axis_index("c")
    s = lax.axis_index("s")
    pltpu.sync_copy(zeros_hbm, zacc.at[pl.ds(s * RPT, RPT)])
    plsc.subcore_barrier()

    def run(y_hbm):
        def body(i, carry):
            base = s * EPT + i * CH
            pltpu.sync_copy(src_hbm.at[pl.ds(base, CH)], src_v)
            pltpu.sync_copy(dst_hbm.at[pl.ds(base, CH)], dst_v)
            pltpu.async_copy(y_hbm.at[src_v], rows_v, sem).wait()
            pltpu.sync_copy(rows_v, zacc.at[dst_v], add=True)
            return carry

        lax.fori_loop(0, NCH, body, 0)

    pl.when(c == 0)(lambda: run(ylo))
    pl.when(c != 0)(lambda: run(yhi))
    plsc.subcore_barrier()
    rows = pl.ds(s * RPT, RPT)
    pl.when(c == 0)(lambda: pltpu.sync_copy(zacc.at[rows], zlo_out.at[rows]))
    pl.when(c != 0)(lambda: pltpu.sync_copy(zacc.at[rows], zhi_out.at[rows]))


def _sc_scatter(ylo, yhi, src, dst):
    zeros = jnp.zeros((RPT, H), jnp.float32)
    return pl.kernel(
        _scatter_body,
        out_type=[jax.ShapeDtypeStruct((NP, H), jnp.float32),
                  jax.ShapeDtypeStruct((NP, H), jnp.float32)],
        mesh=_mesh(),
        scratch_types=[
            pltpu.VMEM_SHARED((NP, H), jnp.float32),
            pltpu.VMEM((CH,), jnp.int32),
            pltpu.VMEM((CH,), jnp.int32),
            pltpu.VMEM((CH, H), jnp.float32),
            pltpu.SemaphoreType.DMA,
        ],
    )(ylo, yhi, src, dst, zeros)


BR = 400
GRID = N // BR


def _dinv_from(dc_blk):
    deg = dc_blk[0, :, 0] + dc_blk[1, :, 0] + 1.0
    return lax.rsqrt(jnp.maximum(deg, 1e-12))


def _mm1_body(x_ref, w_ref, dc_ref, ylo_ref, yhi_ref):
    xw = jnp.dot(x_ref[...], w_ref[...], preferred_element_type=jnp.float32)
    dinv = _dinv_from(dc_ref[...])
    y = xw * dinv[:, None]
    ylo_ref[...] = y[:, :H]
    yhi_ref[...] = y[:, H:]


def _tc_mm1(x, W1, dcount):
    return pl.pallas_call(
        _mm1_body,
        grid=(GRID,),
        in_specs=[
            pl.BlockSpec((BR, D), lambda i: (i, 0)),
            pl.BlockSpec((D, D), lambda i: (0, 0)),
            pl.BlockSpec((2, BR, H), lambda i: (0, i, 0)),
        ],
        out_specs=[pl.BlockSpec((BR, H), lambda i: (i, 0)),
                   pl.BlockSpec((BR, H), lambda i: (i, 0))],
        out_shape=[jax.ShapeDtypeStruct((N, H), jnp.float32),
                   jax.ShapeDtypeStruct((N, H), jnp.float32)],
    )(x, W1, dcount)


def _mm2_body(zlo_ref, zhi_ref, ylo_ref, yhi_ref, dc_ref, b_ref, w_ref,
              olo_ref, ohi_ref):
    z = jnp.concatenate([zlo_ref[...], zhi_ref[...]], axis=1)
    y = jnp.concatenate([ylo_ref[...], yhi_ref[...]], axis=1)
    dinv = _dinv_from(dc_ref[...])
    h = jnp.maximum(dinv[:, None] * (z + y) + b_ref[...][None, :], 0.0)
    xw = jnp.dot(h, w_ref[...], preferred_element_type=jnp.float32)
    y2 = xw * dinv[:, None]
    olo_ref[...] = y2[:, :H]
    ohi_ref[...] = y2[:, H:]


def _tc_mm2(zlo, zhi, ylo, yhi, dcount, b1, W2):
    return pl.pallas_call(
        _mm2_body,
        grid=(GRID,),
        in_specs=[
            pl.BlockSpec((BR, H), lambda i: (i, 0)),
            pl.BlockSpec((BR, H), lambda i: (i, 0)),
            pl.BlockSpec((BR, H), lambda i: (i, 0)),
            pl.BlockSpec((BR, H), lambda i: (i, 0)),
            pl.BlockSpec((2, BR, H), lambda i: (0, i, 0)),
            pl.BlockSpec((D,), lambda i: (0,)),
            pl.BlockSpec((D, D), lambda i: (0, 0)),
        ],
        out_specs=[pl.BlockSpec((BR, H), lambda i: (i, 0)),
                   pl.BlockSpec((BR, H), lambda i: (i, 0))],
        out_shape=[jax.ShapeDtypeStruct((N, H), jnp.float32),
                   jax.ShapeDtypeStruct((N, H), jnp.float32)],
    )(zlo, zhi, ylo, yhi, dcount, b1, W2)


def _mm3_body(zlo_ref, zhi_ref, ylo_ref, yhi_ref, dc_ref, b_ref, wc_ref,
              bc_ref, out_ref):
    z = jnp.concatenate([zlo_ref[...], zhi_ref[...]], axis=1)
    y = jnp.concatenate([ylo_ref[...], yhi_ref[...]], axis=1)
    dinv = _dinv_from(dc_ref[...])
    h = jnp.maximum(dinv[:, None] * (z + y) + b_ref[...][None, :], 0.0)
    out_ref[...] = (jnp.dot(h, wc_ref[...], preferred_element_type=jnp.float32)
                    + bc_ref[...][None, :])


def _tc_mm3(zlo, zhi, ylo, yhi, dcount, b2, Wc, bc):
    return pl.pallas_call(
        _mm3_body,
        grid=(GRID,),
        in_specs=[
            pl.BlockSpec((BR, H), lambda i: (i, 0)),
            pl.BlockSpec((BR, H), lambda i: (i, 0)),
            pl.BlockSpec((BR, H), lambda i: (i, 0)),
            pl.BlockSpec((BR, H), lambda i: (i, 0)),
            pl.BlockSpec((2, BR, H), lambda i: (0, i, 0)),
            pl.BlockSpec((D,), lambda i: (0,)),
            pl.BlockSpec((D, 2), lambda i: (0, 0)),
            pl.BlockSpec((2,), lambda i: (0,)),
        ],
        out_specs=pl.BlockSpec((BR, 2), lambda i: (i, 0)),
        out_shape=jax.ShapeDtypeStruct((N, 2), jnp.float32),
    )(zlo, zhi, ylo, yhi, dcount, b2, Wc, bc)


def kernel(x, edge_index, W1, b1, W2, b2, Wc, bc):
    src = edge_index[0].astype(jnp.int32)
    dst = edge_index[1].astype(jnp.int32)
    dcount = _sc_degree(dst)
    y1lo, y1hi = _tc_mm1(x, W1, dcount)
    z1lo, z1hi = _sc_scatter(y1lo, y1hi, src, dst)
    y2lo, y2hi = _tc_mm2(z1lo, z1hi, y1lo, y1hi, dcount, b1, W2)
    z2lo, z2hi = _sc_scatter(y2lo, y2hi, src, dst)
    return _tc_mm3(z2lo, z2hi, y2lo, y2hi, dcount, b2, Wc, bc)
